# baseline (speedup 1.0000x reference)
import functools
import math

import jax
import jax.numpy as jnp
import numpy as np
from jax import lax
from jax.experimental import pallas as pl
from jax.experimental.pallas import tpu as pltpu
from jax.experimental.pallas import tpu_sc as plsc

_B, _S, _D, _H = 2, 2048, 768, 12
_DH = _D // _H
_BH = _B * _H
_RAND = 5 * int(np.ceil(np.log(_S)))
_TOP = 5 * int(np.log(_S))
_SCALE = 1.0 / math.sqrt(_DH)
_BLK = 256
_NBLK = _S // _BLK

def _threefry_pair(keypair, x0, x1):
    rot1 = (13, 15, 26, 6)
    rot2 = (17, 29, 16, 24)

    def rotl(x, r):
        return (x << np.uint32(r)) | (x >> np.uint32(32 - r))

    x0 = x0.astype(np.uint32).copy()
    x1 = x1.astype(np.uint32).copy()
    ks0, ks1 = np.uint32(keypair[0]), np.uint32(keypair[1])
    ks2 = ks0 ^ ks1 ^ np.uint32(0x1BD11BDA)
    sched = [(rot1, ks1, ks2), (rot2, ks2, ks0), (rot1, ks0, ks1),
             (rot2, ks1, ks2), (rot1, ks2, ks0)]
    with np.errstate(over="ignore"):
        x0 = x0 + ks0
        x1 = x1 + ks1
        for i, (rots, a0, a1) in enumerate(sched):
            for r in rots:
                x0 = x0 + x1
                x1 = rotl(x1, r) ^ x0
            x0 = x0 + a0
            x1 = x1 + a1 + np.uint32(i + 1)
    return x0, x1


def _rand_index():
    root = (np.uint32(0), np.uint32(42))
    z = np.zeros(2, np.uint32)
    b1, b2 = _threefry_pair(root, z, np.arange(2, dtype=np.uint32))
    child = (b1[1], b2[1])
    n = _S * _RAND
    o1, o2 = _threefry_pair(child, np.zeros(n, np.uint32),
                            np.arange(n, dtype=np.uint32))
    bits = o1 ^ o2
    return (bits % np.uint32(_S)).astype(np.int32).reshape(_S, _RAND)


_CONSTS: list = []


def _consts():
    if not _CONSTS:
        ri = _rand_index()
        cnt_T = np.zeros((_S, _S), np.int8)
        np.add.at(cnt_T, (ri.ravel(), np.repeat(np.arange(_S), _RAND)), 1)
        ltri = np.tril(np.ones((_BLK, _BLK), np.float32))
        _CONSTS.append((cnt_T, ltri))
    return _CONSTS[0]


def _linear_kern(x_ref, w_ref, b_ref, o_ref):
    o_ref[0, 0] = (
        jnp.dot(x_ref[0, 0], w_ref[0], preferred_element_type=jnp.float32)
        + b_ref[0]
    )


def _linear(xs, ws, bs, n_stack, sb=512):
    return pl.pallas_call(
        _linear_kern,
        grid=(n_stack, _B, _S // sb),
        in_specs=[
            pl.BlockSpec((1, 1, sb, _D), lambda i, b, s: (i, b, s, 0)),
            pl.BlockSpec((1, _D, _D), lambda i, b, s: (i, 0, 0)),
            pl.BlockSpec((1, 1, _D), lambda i, b, s: (i, 0, 0)),
        ],
        out_specs=pl.BlockSpec((1, 1, sb, _D), lambda i, b, s: (i, b, s, 0)),
        out_shape=jax.ShapeDtypeStruct((n_stack, _B, _S, _D), jnp.float32),
    )(xs, ws, bs)


_NT = (((1,), (1,)), ((), ()))


def _attn_kern(q_ref, k_ref, v_ref, cnt_ref, upd_ref, io_ref, qi_ref):
    def blk_body(r, carry):
        smax, ssum = carry
        kb = k_ref[0, 0, pl.ds(r * _BLK, _BLK), :]
        at = jax.lax.dot_general(kb, q_ref[0, 0], _NT,
                                 preferred_element_type=jnp.float32)
        cf = cnt_ref[pl.ds(r * _BLK, _BLK), :].astype(jnp.float32)
        masked = jnp.where(cf > 0.0, at, -1e30)
        smax = jnp.maximum(smax, jnp.max(masked, axis=0, keepdims=True))
        ssum = ssum + jnp.sum(at * cf, axis=0, keepdims=True)
        return smax, ssum

    init = (jnp.full((1, _S), -1e30, jnp.float32), jnp.zeros((1, _S), jnp.float32))
    smax, ssum = jax.lax.fori_loop(0, _NBLK, blk_body, init)
    disc = smax - ssum / _S

    lane = jax.lax.broadcasted_iota(jnp.int32, (1, _S), 1)
    lane128 = jax.lax.broadcasted_iota(jnp.int32, (1, 128), 1)
    qi_ref[...] = jnp.zeros((_RAND, _DH), jnp.float32)
    gbase = pl.program_id(0) * _S

    def top_body(i, carry):
        dcur, ivec = carry
        m = jnp.max(dcur)
        idx = jnp.min(jnp.where(dcur == m, lane, _S))
        qi_ref[pl.ds(i, 1), :] = q_ref[0, 0, pl.ds(idx, 1), :]
        ivec = jnp.where(lane128 == i, gbase + idx, ivec)
        return jnp.where(lane == idx, -3e38, dcur), ivec

    _, ivec = jax.lax.fori_loop(
        0, _TOP, top_body, (disc, jnp.zeros((1, 128), jnp.int32)))
    g0 = jnp.min(jnp.where(lane128 == 0, ivec, jnp.int32(2**30)))
    ivec = jnp.where(lane128 >= _TOP, g0, ivec)
    io_ref[0] = ivec[:, :_RAND]

    qi_ref[pl.ds(_TOP, _RAND - _TOP), :] = jnp.broadcast_to(
        qi_ref[pl.ds(0, 1), :], (_RAND - _TOP, _DH))

    qk = jax.lax.dot_general(qi_ref[...], k_ref[0, 0], _NT,
                             preferred_element_type=jnp.float32) * _SCALE
    m = jnp.max(qk, axis=1, keepdims=True)
    e = jnp.exp(qk - m)
    p = e / jnp.sum(e, axis=1, keepdims=True)
    upd_ref[0] = jnp.dot(p, v_ref[0, 0], preferred_element_type=jnp.float32)


def _attn(qkvh, cntT):
    head_spec = lambda i: pl.BlockSpec(
        (1, 1, _S, _DH), lambda g, i=i: (i, g, 0, 0))
    return pl.pallas_call(
        _attn_kern,
        grid=(_BH,),
        in_specs=[
            head_spec(0),
            head_spec(1),
            head_spec(2),
            pl.BlockSpec((_S, _S), lambda g: (0, 0)),
        ],
        out_specs=[
            pl.BlockSpec((1, _RAND, _DH), lambda g: (g, 0, 0)),
            pl.BlockSpec((1, 1, _RAND), lambda g: (g, 0, 0)),
        ],
        out_shape=[
            jax.ShapeDtypeStruct((_BH, _RAND, _DH), jnp.float32),
            jax.ShapeDtypeStruct((_BH, 1, _RAND), jnp.int32),
        ],
        scratch_shapes=[
            pltpu.VMEM((_RAND, _DH), jnp.float32),
        ],
    )(qkvh, qkvh, qkvh, cntT)


_CHUNK = 512
_NCH = _S // _CHUNK


def _sc_cumsum_scatter(qkvh, upd, idx):
    mesh = plsc.VectorSubcoreMesh(core_axis_name="c", subcore_axis_name="s")

    @functools.partial(
        pl.kernel,
        mesh=mesh,
        out_type=jax.ShapeDtypeStruct((_BH * _S, _DH), jnp.float32),
        scratch_types=[
            pltpu.VMEM((_CHUNK, _DH), jnp.float32),
            pltpu.VMEM((16, _DH), jnp.float32),
            pltpu.VMEM((1024,), jnp.int32),
        ],
        compiler_params=pltpu.CompilerParams(use_tc_tiling_on_sc=False),
    )
    def cum_kernel(qkvh_hbm, upd_hbm, idx_hbm, out_hbm, chunk_v, rows_v, idx_v):
        wid = lax.axis_index("s") * 2 + lax.axis_index("c")

        @pl.when(wid < _BH)
        def _():
            bh = wid
            pltpu.sync_copy(idx_hbm, idx_v)
            carry = tuple(jnp.zeros((16,), jnp.float32) for _ in range(4))
            for ch in range(_NCH):
                pltpu.sync_copy(
                    qkvh_hbm.at[2, bh, pl.ds(ch * _CHUNK, _CHUNK), :],
                    chunk_v)

                def body(i, acc):
                    outs = []
                    for d in range(4):
                        a = acc[d] + chunk_v[i, pl.ds(d * 16, 16)]
                        chunk_v[i, pl.ds(d * 16, 16)] = a
                        outs.append(a)
                    return tuple(outs)

                carry = jax.lax.fori_loop(0, _CHUNK, body, carry)
                pltpu.sync_copy(
                    chunk_v,
                    out_hbm.at[pl.ds((bh * _NCH + ch) * _CHUNK, _CHUNK), :])

            for gi in range(3):
                idx16 = idx_v[pl.ds(bh * _RAND + gi * 16, 16)]
                pltpu.sync_copy(
                    upd_hbm.at[pl.ds(bh * _RAND + gi * 16, 16), :], rows_v)
                pltpu.sync_copy(rows_v, out_hbm.at[idx16])

    return cum_kernel(qkvh, upd, idx)


def kernel(queries, keys, values, Wq, bq, Wk, bk, Wv, bv, Wo, bo):
    xs = jnp.stack([queries, keys, values])
    ws = jnp.stack([Wq.T, Wk.T, Wv.T])
    bs = jnp.stack([bq, bk, bv])[:, None, :]
    qkv = _linear(xs, ws, bs, 3)

    qkvh = (
        qkv.reshape(3, _B, _S, _H, _DH)
        .transpose(0, 1, 3, 2, 4)
        .reshape(3, _BH, _S, _DH)
    )
    cnt_T, _ = _consts()
    upd, idx = _attn(qkvh, jnp.asarray(cnt_T))

    upd2d = upd.reshape(_BH * _RAND, _DH)
    idx1d = idx.reshape(_BH * _RAND)
    npad = 1024 - _BH * _RAND
    upd_pad = jnp.concatenate(
        [upd2d, jnp.broadcast_to(upd2d[:1], (npad, _DH))])
    idx_pad = jnp.concatenate(
        [idx1d, jnp.broadcast_to(idx1d[:1], (npad,))])
    vc = _sc_cumsum_scatter(qkvh, upd_pad, idx_pad)

    vc2 = (
        vc.reshape(_B, _H, _S, _DH)
        .transpose(0, 2, 1, 3)
        .reshape(1, _B, _S, _D)
    )
    out = _linear(vc2, Wo.T[None], bo[None, None, :], 1)
    return out[0]


# speedup vs baseline: 1.0059x; 1.0059x over previous
import functools
import math

import jax
import jax.numpy as jnp
import numpy as np
from jax import lax
from jax.experimental import pallas as pl
from jax.experimental.pallas import tpu as pltpu
from jax.experimental.pallas import tpu_sc as plsc

_B, _S, _D, _H = 2, 2048, 768, 12
_DH = _D // _H
_BH = _B * _H
_RAND = 5 * int(np.ceil(np.log(_S)))
_TOP = 5 * int(np.log(_S))
_SCALE = 1.0 / math.sqrt(_DH)
_BLK = 256
_NBLK = _S // _BLK

def _threefry_pair(keypair, x0, x1):
    rot1 = (13, 15, 26, 6)
    rot2 = (17, 29, 16, 24)

    def rotl(x, r):
        return (x << np.uint32(r)) | (x >> np.uint32(32 - r))

    x0 = x0.astype(np.uint32).copy()
    x1 = x1.astype(np.uint32).copy()
    ks0, ks1 = np.uint32(keypair[0]), np.uint32(keypair[1])
    ks2 = ks0 ^ ks1 ^ np.uint32(0x1BD11BDA)
    sched = [(rot1, ks1, ks2), (rot2, ks2, ks0), (rot1, ks0, ks1),
             (rot2, ks1, ks2), (rot1, ks2, ks0)]
    with np.errstate(over="ignore"):
        x0 = x0 + ks0
        x1 = x1 + ks1
        for i, (rots, a0, a1) in enumerate(sched):
            for r in rots:
                x0 = x0 + x1
                x1 = rotl(x1, r) ^ x0
            x0 = x0 + a0
            x1 = x1 + a1 + np.uint32(i + 1)
    return x0, x1


def _rand_index():
    root = (np.uint32(0), np.uint32(42))
    z = np.zeros(2, np.uint32)
    b1, b2 = _threefry_pair(root, z, np.arange(2, dtype=np.uint32))
    child = (b1[1], b2[1])
    n = _S * _RAND
    o1, o2 = _threefry_pair(child, np.zeros(n, np.uint32),
                            np.arange(n, dtype=np.uint32))
    bits = o1 ^ o2
    return (bits % np.uint32(_S)).astype(np.int32).reshape(_S, _RAND)


_CONSTS: list = []


def _consts():
    if not _CONSTS:
        ri = _rand_index()
        cnt_T = np.zeros((_S, _S), np.int8)
        np.add.at(cnt_T, (ri.ravel(), np.repeat(np.arange(_S), _RAND)), 1)
        ltri = np.tril(np.ones((_BLK, _BLK), np.float32))
        _CONSTS.append((cnt_T, ltri))
    return _CONSTS[0]


def _linear_kern(x_ref, w_ref, b_ref, o_ref):
    o_ref[0, 0] = (
        jnp.dot(x_ref[0, 0], w_ref[0], preferred_element_type=jnp.float32)
        + b_ref[0]
    )


def _linear(xs, ws, bs, n_stack, sb=512):
    return pl.pallas_call(
        _linear_kern,
        grid=(n_stack, _B, _S // sb),
        in_specs=[
            pl.BlockSpec((1, 1, sb, _D), lambda i, b, s: (i, b, s, 0)),
            pl.BlockSpec((1, _D, _D), lambda i, b, s: (i, 0, 0)),
            pl.BlockSpec((1, 1, _D), lambda i, b, s: (i, 0, 0)),
        ],
        out_specs=pl.BlockSpec((1, 1, sb, _D), lambda i, b, s: (i, b, s, 0)),
        out_shape=jax.ShapeDtypeStruct((n_stack, _B, _S, _D), jnp.float32),
    )(xs, ws, bs)


_NT = (((1,), (1,)), ((), ()))


def _attn_kern(q_ref, k_ref, v_ref, cnt_ref, upd_ref, io_ref, qi_ref):
    def blk_body(r, carry):
        smax, ssum = carry
        kb = k_ref[0, 0, pl.ds(r * _BLK, _BLK), :]
        at = jax.lax.dot_general(kb, q_ref[0, 0], _NT,
                                 preferred_element_type=jnp.float32)
        cf = cnt_ref[pl.ds(r * _BLK, _BLK), :].astype(jnp.float32)
        masked = jnp.where(cf > 0.0, at, -1e30)
        smax = jnp.maximum(smax, jnp.max(masked, axis=0, keepdims=True))
        ssum = ssum + jnp.sum(at * cf, axis=0, keepdims=True)
        return smax, ssum

    init = (jnp.full((1, _S), -1e30, jnp.float32), jnp.zeros((1, _S), jnp.float32))
    smax, ssum = jax.lax.fori_loop(0, _NBLK, blk_body, init)
    disc = smax - ssum / _S

    lane = jax.lax.broadcasted_iota(jnp.int32, (1, _S), 1)
    lane128 = jax.lax.broadcasted_iota(jnp.int32, (1, 128), 1)
    qi_ref[...] = jnp.zeros((_RAND, _DH), jnp.float32)

    def top_body(i, carry):
        dcur, ivec = carry
        m = jnp.max(dcur)
        idx = jnp.min(jnp.where(dcur == m, lane, _S))
        qi_ref[pl.ds(i, 1), :] = q_ref[0, 0, pl.ds(idx, 1), :]
        ivec = jnp.where(lane128 == i, idx, ivec)
        return jnp.where(lane == idx, -3e38, dcur), ivec

    _, ivec = jax.lax.fori_loop(
        0, _TOP, top_body, (disc, jnp.zeros((1, 128), jnp.int32)))
    io_ref[0] = ivec[:, :_RAND]

    qi_ref[pl.ds(_TOP, _RAND - _TOP), :] = jnp.broadcast_to(
        qi_ref[pl.ds(0, 1), :], (_RAND - _TOP, _DH))

    qk = jax.lax.dot_general(qi_ref[...], k_ref[0, 0], _NT,
                             preferred_element_type=jnp.float32) * _SCALE
    m = jnp.max(qk, axis=1, keepdims=True)
    e = jnp.exp(qk - m)
    p = e / jnp.sum(e, axis=1, keepdims=True)
    upd_ref[0] = jnp.dot(p, v_ref[0, 0], preferred_element_type=jnp.float32)


def _attn(qkvh, cntT):
    head_spec = lambda i: pl.BlockSpec(
        (1, 1, _S, _DH), lambda g, i=i: (i, g, 0, 0))
    return pl.pallas_call(
        _attn_kern,
        grid=(_BH,),
        in_specs=[
            head_spec(0),
            head_spec(1),
            head_spec(2),
            pl.BlockSpec((_S, _S), lambda g: (0, 0)),
        ],
        out_specs=[
            pl.BlockSpec((1, _RAND, _DH), lambda g: (g, 0, 0)),
            pl.BlockSpec((1, 1, _RAND), lambda g: (g, 0, 0)),
        ],
        out_shape=[
            jax.ShapeDtypeStruct((_BH, _RAND, _DH), jnp.float32),
            jax.ShapeDtypeStruct((_BH, 1, _RAND), jnp.int32),
        ],
        scratch_shapes=[
            pltpu.VMEM((_RAND, _DH), jnp.float32),
        ],
    )(qkvh, qkvh, qkvh, cntT)


_CHUNK = 512
_NCH = _S // _CHUNK


def _sc_cumsum(qkvh):
    mesh = plsc.VectorSubcoreMesh(core_axis_name="c", subcore_axis_name="s")

    @functools.partial(
        pl.kernel,
        mesh=mesh,
        out_type=jax.ShapeDtypeStruct((_BH * _S, _DH), jnp.float32),
        scratch_types=[
            pltpu.VMEM((_CHUNK, _DH), jnp.float32),
        ],
        compiler_params=pltpu.CompilerParams(use_tc_tiling_on_sc=False),
    )
    def cum_kernel(qkvh_hbm, out_hbm, chunk_v):
        wid = lax.axis_index("s") * 2 + lax.axis_index("c")

        @pl.when(wid < _BH)
        def _():
            bh = wid
            carry = tuple(jnp.zeros((16,), jnp.float32) for _ in range(4))
            for ch in range(_NCH):
                pltpu.sync_copy(
                    qkvh_hbm.at[2, bh, pl.ds(ch * _CHUNK, _CHUNK), :],
                    chunk_v)

                def body(i, acc):
                    outs = []
                    for d in range(4):
                        a = acc[d] + chunk_v[i, pl.ds(d * 16, 16)]
                        chunk_v[i, pl.ds(d * 16, 16)] = a
                        outs.append(a)
                    return tuple(outs)

                carry = jax.lax.fori_loop(0, _CHUNK, body, carry)
                pltpu.sync_copy(
                    chunk_v,
                    out_hbm.at[pl.ds((bh * _NCH + ch) * _CHUNK, _CHUNK), :])

    return cum_kernel(qkvh)


def _scat_kern(idx_sref, vc_ref, upd_ref, o_ref):
    o_ref[...] = vc_ref[...]
    g = pl.program_id(0)

    def body(i, c):
        s = idx_sref[g, i]
        o_ref[0, pl.ds(s, 1), :] = upd_ref[0, pl.ds(i, 1), :]
        return c

    jax.lax.fori_loop(0, _TOP, body, 0)


def _tc_scatter(idx, vc, upd):
    grid_spec = pltpu.PrefetchScalarGridSpec(
        num_scalar_prefetch=1,
        grid=(_BH,),
        in_specs=[
            pl.BlockSpec((1, _S, _DH), lambda g, sref: (g, 0, 0)),
            pl.BlockSpec((1, _RAND, _DH), lambda g, sref: (g, 0, 0)),
        ],
        out_specs=pl.BlockSpec((1, _S, _DH), lambda g, sref: (g, 0, 0)),
    )
    return pl.pallas_call(
        _scat_kern,
        grid_spec=grid_spec,
        out_shape=jax.ShapeDtypeStruct((_BH, _S, _DH), jnp.float32),
    )(idx, vc, upd)


def kernel(queries, keys, values, Wq, bq, Wk, bk, Wv, bv, Wo, bo):
    xs = jnp.stack([queries, keys, values])
    ws = jnp.stack([Wq.T, Wk.T, Wv.T])
    bs = jnp.stack([bq, bk, bv])[:, None, :]
    qkv = _linear(xs, ws, bs, 3)

    qkvh = (
        qkv.reshape(3, _B, _S, _H, _DH)
        .transpose(0, 1, 3, 2, 4)
        .reshape(3, _BH, _S, _DH)
    )
    cnt_T, _ = _consts()
    vcum = _sc_cumsum(qkvh)
    upd, idx = _attn(qkvh, jnp.asarray(cnt_T))
    vc = _tc_scatter(
        idx.reshape(_BH, _RAND), vcum.reshape(_BH, _S, _DH), upd)

    vc2 = (
        vc.reshape(_B, _H, _S, _DH)
        .transpose(0, 2, 1, 3)
        .reshape(1, _B, _S, _D)
    )
    out = _linear(vc2, Wo.T[None], bo[None, None, :], 1)
    return out[0]


# speedup vs baseline: 1.0061x; 1.0001x over previous
import functools
import math

import jax
import jax.numpy as jnp
import numpy as np
from jax import lax
from jax.experimental import pallas as pl
from jax.experimental.pallas import tpu as pltpu
from jax.experimental.pallas import tpu_sc as plsc

_B, _S, _D, _H = 2, 2048, 768, 12
_DH = _D // _H
_BH = _B * _H
_RAND = 5 * int(np.ceil(np.log(_S)))
_TOP = 5 * int(np.log(_S))
_SCALE = 1.0 / math.sqrt(_DH)
_BLK = 256
_NBLK = _S // _BLK

def _threefry_pair(keypair, x0, x1):
    rot1 = (13, 15, 26, 6)
    rot2 = (17, 29, 16, 24)

    def rotl(x, r):
        return (x << np.uint32(r)) | (x >> np.uint32(32 - r))

    x0 = x0.astype(np.uint32).copy()
    x1 = x1.astype(np.uint32).copy()
    ks0, ks1 = np.uint32(keypair[0]), np.uint32(keypair[1])
    ks2 = ks0 ^ ks1 ^ np.uint32(0x1BD11BDA)
    sched = [(rot1, ks1, ks2), (rot2, ks2, ks0), (rot1, ks0, ks1),
             (rot2, ks1, ks2), (rot1, ks2, ks0)]
    with np.errstate(over="ignore"):
        x0 = x0 + ks0
        x1 = x1 + ks1
        for i, (rots, a0, a1) in enumerate(sched):
            for r in rots:
                x0 = x0 + x1
                x1 = rotl(x1, r) ^ x0
            x0 = x0 + a0
            x1 = x1 + a1 + np.uint32(i + 1)
    return x0, x1


def _rand_index():
    root = (np.uint32(0), np.uint32(42))
    z = np.zeros(2, np.uint32)
    b1, b2 = _threefry_pair(root, z, np.arange(2, dtype=np.uint32))
    child = (b1[1], b2[1])
    n = _S * _RAND
    o1, o2 = _threefry_pair(child, np.zeros(n, np.uint32),
                            np.arange(n, dtype=np.uint32))
    bits = o1 ^ o2
    return (bits % np.uint32(_S)).astype(np.int32).reshape(_S, _RAND)


_CONSTS: list = []


def _consts():
    if not _CONSTS:
        ri = _rand_index()
        cnt_T = np.zeros((_S, _S), np.int8)
        np.add.at(cnt_T, (ri.ravel(), np.repeat(np.arange(_S), _RAND)), 1)
        ltri = np.tril(np.ones((_BLK, _BLK), np.float32))
        _CONSTS.append((cnt_T, ltri))
    return _CONSTS[0]


def _linear_kern(x_ref, w_ref, b_ref, o_ref):
    o_ref[0, 0] = (
        jnp.dot(x_ref[0, 0], w_ref[0], preferred_element_type=jnp.float32)
        + b_ref[0]
    )


def _linear(xs, ws, bs, n_stack, sb=512):
    return pl.pallas_call(
        _linear_kern,
        grid=(n_stack, _B, _S // sb),
        in_specs=[
            pl.BlockSpec((1, 1, sb, _D), lambda i, b, s: (i, b, s, 0)),
            pl.BlockSpec((1, _D, _D), lambda i, b, s: (i, 0, 0)),
            pl.BlockSpec((1, 1, _D), lambda i, b, s: (i, 0, 0)),
        ],
        out_specs=pl.BlockSpec((1, 1, sb, _D), lambda i, b, s: (i, b, s, 0)),
        out_shape=jax.ShapeDtypeStruct((n_stack, _B, _S, _D), jnp.float32),
    )(xs, ws, bs)


_NT = (((1,), (1,)), ((), ()))


def _attn_kern(q_ref, k_ref, v_ref, vcum_ref, cnt_ref, o_ref, qi_ref, idx_ref):
    def blk_body(r, carry):
        smax, ssum = carry
        kb = k_ref[0, 0, pl.ds(r * _BLK, _BLK), :]
        at = jax.lax.dot_general(kb, q_ref[0, 0], _NT,
                                 preferred_element_type=jnp.float32)
        cf = cnt_ref[pl.ds(r * _BLK, _BLK), :].astype(jnp.float32)
        masked = jnp.where(cf > 0.0, at, -1e30)
        smax = jnp.maximum(smax, jnp.max(masked, axis=0, keepdims=True))
        ssum = ssum + jnp.sum(at * cf, axis=0, keepdims=True)
        return smax, ssum

    init = (jnp.full((1, _S), -1e30, jnp.float32), jnp.zeros((1, _S), jnp.float32))
    smax, ssum = jax.lax.fori_loop(0, _NBLK, blk_body, init)
    disc = smax - ssum / _S

    lane = jax.lax.broadcasted_iota(jnp.int32, (1, _S), 1)
    qi_ref[...] = jnp.zeros((_RAND, _DH), jnp.float32)

    def top_body(i, dcur):
        m = jnp.max(dcur)
        idx = jnp.min(jnp.where(dcur == m, lane, _S))
        idx_ref[i] = idx
        qi_ref[pl.ds(i, 1), :] = q_ref[0, 0, pl.ds(idx, 1), :]
        return jnp.where(lane == idx, -3e38, dcur)

    jax.lax.fori_loop(0, _TOP, top_body, disc)

    qi_ref[pl.ds(_TOP, _RAND - _TOP), :] = jnp.broadcast_to(
        qi_ref[pl.ds(0, 1), :], (_RAND - _TOP, _DH))

    qk = jax.lax.dot_general(qi_ref[...], k_ref[0, 0], _NT,
                             preferred_element_type=jnp.float32) * _SCALE
    m = jnp.max(qk, axis=1, keepdims=True)
    e = jnp.exp(qk - m)
    p = e / jnp.sum(e, axis=1, keepdims=True)
    upd = jnp.dot(p, v_ref[0, 0], preferred_element_type=jnp.float32)

    o_ref[0] = vcum_ref[0]
    qi_ref[...] = upd

    def scat_body(i, c):
        s = idx_ref[i]
        o_ref[0, pl.ds(s, 1), :] = qi_ref[pl.ds(i, 1), :]
        return c

    jax.lax.fori_loop(0, _TOP, scat_body, 0)


def _attn(qkvh, vcum, cntT):
    head_spec = lambda i: pl.BlockSpec(
        (1, 1, _S, _DH), lambda g, i=i: (i, g, 0, 0))
    return pl.pallas_call(
        _attn_kern,
        grid=(_BH,),
        in_specs=[
            head_spec(0),
            head_spec(1),
            head_spec(2),
            pl.BlockSpec((1, _S, _DH), lambda g: (g, 0, 0)),
            pl.BlockSpec((_S, _S), lambda g: (0, 0)),
        ],
        out_specs=pl.BlockSpec((1, _S, _DH), lambda g: (g, 0, 0)),
        out_shape=jax.ShapeDtypeStruct((_BH, _S, _DH), jnp.float32),
        scratch_shapes=[
            pltpu.VMEM((_RAND, _DH), jnp.float32),
            pltpu.SMEM((_RAND,), jnp.int32),
        ],
    )(qkvh, qkvh, qkvh, vcum, cntT)


_CHUNK = 512
_NCH = _S // _CHUNK


def _sc_cumsum(qkvh):
    mesh = plsc.VectorSubcoreMesh(core_axis_name="c", subcore_axis_name="s")

    @functools.partial(
        pl.kernel,
        mesh=mesh,
        out_type=jax.ShapeDtypeStruct((_BH * _S, _DH), jnp.float32),
        scratch_types=[
            pltpu.VMEM((_CHUNK, _DH), jnp.float32),
        ],
        compiler_params=pltpu.CompilerParams(use_tc_tiling_on_sc=False),
    )
    def cum_kernel(qkvh_hbm, out_hbm, chunk_v):
        wid = lax.axis_index("s") * 2 + lax.axis_index("c")

        @pl.when(wid < _BH)
        def _():
            bh = wid
            carry = tuple(jnp.zeros((16,), jnp.float32) for _ in range(4))
            for ch in range(_NCH):
                pltpu.sync_copy(
                    qkvh_hbm.at[2, bh, pl.ds(ch * _CHUNK, _CHUNK), :],
                    chunk_v)

                def body(i, acc):
                    outs = []
                    for d in range(4):
                        a = acc[d] + chunk_v[i, pl.ds(d * 16, 16)]
                        chunk_v[i, pl.ds(d * 16, 16)] = a
                        outs.append(a)
                    return tuple(outs)

                carry = jax.lax.fori_loop(0, _CHUNK, body, carry)
                pltpu.sync_copy(
                    chunk_v,
                    out_hbm.at[pl.ds((bh * _NCH + ch) * _CHUNK, _CHUNK), :])

    return cum_kernel(qkvh)


def kernel(queries, keys, values, Wq, bq, Wk, bk, Wv, bv, Wo, bo):
    xs = jnp.stack([queries, keys, values])
    ws = jnp.stack([Wq.T, Wk.T, Wv.T])
    bs = jnp.stack([bq, bk, bv])[:, None, :]
    qkv = _linear(xs, ws, bs, 3)

    qkvh = (
        qkv.reshape(3, _B, _S, _H, _DH)
        .transpose(0, 1, 3, 2, 4)
        .reshape(3, _BH, _S, _DH)
    )
    cnt_T, _ = _consts()
    vcum = _sc_cumsum(qkvh)
    vc = _attn(qkvh, vcum.reshape(_BH, _S, _DH), jnp.asarray(cnt_T))

    vc2 = (
        vc.reshape(_B, _H, _S, _DH)
        .transpose(0, 2, 1, 3)
        .reshape(1, _B, _S, _D)
    )
    out = _linear(vc2, Wo.T[None], bo[None, None, :], 1)
    return out[0]


# speedup vs baseline: 1.0129x; 1.0068x over previous
import functools
import math

import jax
import jax.numpy as jnp
import numpy as np
from jax import lax
from jax.experimental import pallas as pl
from jax.experimental.pallas import tpu as pltpu
from jax.experimental.pallas import tpu_sc as plsc

_B, _S, _D, _H = 2, 2048, 768, 12
_DH = _D // _H
_BH = _B * _H
_RAND = 5 * int(np.ceil(np.log(_S)))
_TOP = 5 * int(np.log(_S))
_SCALE = 1.0 / math.sqrt(_DH)
_BLK = 256
_NBLK = _S // _BLK

def _threefry_pair(keypair, x0, x1):
    rot1 = (13, 15, 26, 6)
    rot2 = (17, 29, 16, 24)

    def rotl(x, r):
        return (x << np.uint32(r)) | (x >> np.uint32(32 - r))

    x0 = x0.astype(np.uint32).copy()
    x1 = x1.astype(np.uint32).copy()
    ks0, ks1 = np.uint32(keypair[0]), np.uint32(keypair[1])
    ks2 = ks0 ^ ks1 ^ np.uint32(0x1BD11BDA)
    sched = [(rot1, ks1, ks2), (rot2, ks2, ks0), (rot1, ks0, ks1),
             (rot2, ks1, ks2), (rot1, ks2, ks0)]
    with np.errstate(over="ignore"):
        x0 = x0 + ks0
        x1 = x1 + ks1
        for i, (rots, a0, a1) in enumerate(sched):
            for r in rots:
                x0 = x0 + x1
                x1 = rotl(x1, r) ^ x0
            x0 = x0 + a0
            x1 = x1 + a1 + np.uint32(i + 1)
    return x0, x1


def _rand_index():
    root = (np.uint32(0), np.uint32(42))
    z = np.zeros(2, np.uint32)
    b1, b2 = _threefry_pair(root, z, np.arange(2, dtype=np.uint32))
    child = (b1[1], b2[1])
    n = _S * _RAND
    o1, o2 = _threefry_pair(child, np.zeros(n, np.uint32),
                            np.arange(n, dtype=np.uint32))
    bits = o1 ^ o2
    return (bits % np.uint32(_S)).astype(np.int32).reshape(_S, _RAND)


_CONSTS: list = []


def _consts():
    if not _CONSTS:
        ri = _rand_index()
        cnt_T = np.zeros((_S, _S), np.int8)
        np.add.at(cnt_T, (ri.ravel(), np.repeat(np.arange(_S), _RAND)), 1)
        ltri = np.tril(np.ones((_BLK, _BLK), np.float32))
        _CONSTS.append((cnt_T, ltri))
    return _CONSTS[0]


def _linear_kern(x_ref, w_ref, b_ref, o_ref):
    o_ref[0, 0] = (
        jnp.dot(x_ref[0, 0], w_ref[0], preferred_element_type=jnp.float32)
        + b_ref[0]
    )


def _linear(xs, ws, bs, n_stack, sb=512):
    return pl.pallas_call(
        _linear_kern,
        grid=(n_stack, _B, _S // sb),
        in_specs=[
            pl.BlockSpec((1, 1, sb, _D), lambda i, b, s: (i, b, s, 0)),
            pl.BlockSpec((1, _D, _D), lambda i, b, s: (i, 0, 0)),
            pl.BlockSpec((1, 1, _D), lambda i, b, s: (i, 0, 0)),
        ],
        out_specs=pl.BlockSpec((1, 1, sb, _D), lambda i, b, s: (i, b, s, 0)),
        out_shape=jax.ShapeDtypeStruct((n_stack, _B, _S, _D), jnp.float32),
    )(xs, ws, bs)


_NT = (((1,), (1,)), ((), ()))


def _attn_kern(q_ref, k_ref, v_ref, vcum_ref, cnt_ref, o_ref, qi_ref, idx_ref):
    def blk_body(r, carry):
        smax, ssum = carry
        kb = k_ref[0, 0, pl.ds(r * _BLK, _BLK), :]
        at = jax.lax.dot_general(kb, q_ref[0, 0], _NT,
                                 preferred_element_type=jnp.float32)
        cf = cnt_ref[pl.ds(r * _BLK, _BLK), :].astype(jnp.float32)
        masked = jnp.where(cf > 0.0, at, -1e30)
        smax = jnp.maximum(smax, jnp.max(masked, axis=0, keepdims=True))
        ssum = ssum + jnp.sum(at * cf, axis=0, keepdims=True)
        return smax, ssum

    init = (jnp.full((1, _S), -1e30, jnp.float32), jnp.zeros((1, _S), jnp.float32))
    smax, ssum = jax.lax.fori_loop(0, _NBLK, blk_body, init)
    disc = smax - ssum / _S

    lane = jax.lax.broadcasted_iota(jnp.int32, (1, _S), 1)
    qi_ref[...] = jnp.zeros((_RAND, _DH), jnp.float32)

    def top_body(i, dcur):
        m = jnp.max(dcur)
        idx = jnp.min(jnp.where(dcur == m, lane, _S))
        idx_ref[i] = idx
        qi_ref[pl.ds(i, 1), :] = q_ref[0, 0, pl.ds(idx, 1), :]
        return jnp.where(lane == idx, -3e38, dcur)

    jax.lax.fori_loop(0, _TOP, top_body, disc)

    qi_ref[pl.ds(_TOP, _RAND - _TOP), :] = jnp.broadcast_to(
        qi_ref[pl.ds(0, 1), :], (_RAND - _TOP, _DH))

    qk = jax.lax.dot_general(qi_ref[...], k_ref[0, 0], _NT,
                             preferred_element_type=jnp.float32) * _SCALE
    m = jnp.max(qk, axis=1, keepdims=True)
    e = jnp.exp(qk - m)
    p = e / jnp.sum(e, axis=1, keepdims=True)
    upd = jnp.dot(p, v_ref[0, 0], preferred_element_type=jnp.float32)

    o_ref[0] = vcum_ref[0]
    qi_ref[...] = upd

    def scat_body(i, c):
        s = idx_ref[i]
        o_ref[0, pl.ds(s, 1), :] = qi_ref[pl.ds(i, 1), :]
        return c

    jax.lax.fori_loop(0, _TOP, scat_body, 0)


def _attn(qkvh, vcum, cntT):
    head_spec = lambda i: pl.BlockSpec(
        (1, 1, _S, _DH), lambda g, i=i: (i, g, 0, 0))
    return pl.pallas_call(
        _attn_kern,
        grid=(_BH,),
        in_specs=[
            head_spec(0),
            head_spec(1),
            head_spec(2),
            pl.BlockSpec((1, _S, _DH), lambda g: (g, 0, 0)),
            pl.BlockSpec((_S, _S), lambda g: (0, 0)),
        ],
        out_specs=pl.BlockSpec((1, _S, _DH), lambda g: (g, 0, 0)),
        out_shape=jax.ShapeDtypeStruct((_BH, _S, _DH), jnp.float32),
        scratch_shapes=[
            pltpu.VMEM((_RAND, _DH), jnp.float32),
            pltpu.SMEM((_RAND,), jnp.int32),
        ],
    )(qkvh, qkvh, qkvh, vcum, cntT)


_CHUNK = 512
_NCH = _S // _CHUNK


def _sc_cumsum(qkvh):
    mesh = plsc.VectorSubcoreMesh(core_axis_name="c", subcore_axis_name="s")

    @functools.partial(
        pl.kernel,
        mesh=mesh,
        out_type=jax.ShapeDtypeStruct((_BH * _S, _DH), jnp.float32),
        scratch_types=[
            pltpu.VMEM((_CHUNK, _DH), jnp.float32),
        ],
        compiler_params=pltpu.CompilerParams(use_tc_tiling_on_sc=False),
    )
    def cum_kernel(qkvh_hbm, out_hbm, chunk_v):
        wid = lax.axis_index("s") * 2 + lax.axis_index("c")

        @pl.when(wid < _BH)
        def _():
            bh = wid
            carry = tuple(jnp.zeros((16,), jnp.float32) for _ in range(4))
            for ch in range(_NCH):
                pltpu.sync_copy(
                    qkvh_hbm.at[2, bh, pl.ds(ch * _CHUNK, _CHUNK), :],
                    chunk_v)

                def body(i8, acc):
                    for r in range(8):
                        i = i8 * 8 + r
                        outs = []
                        for d in range(4):
                            a = acc[d] + chunk_v[i, pl.ds(d * 16, 16)]
                            chunk_v[i, pl.ds(d * 16, 16)] = a
                            outs.append(a)
                        acc = tuple(outs)
                    return acc

                carry = jax.lax.fori_loop(0, _CHUNK // 8, body, carry)
                pltpu.sync_copy(
                    chunk_v,
                    out_hbm.at[pl.ds((bh * _NCH + ch) * _CHUNK, _CHUNK), :])

    return cum_kernel(qkvh)


def kernel(queries, keys, values, Wq, bq, Wk, bk, Wv, bv, Wo, bo):
    xs = jnp.stack([queries, keys, values])
    ws = jnp.stack([Wq.T, Wk.T, Wv.T])
    bs = jnp.stack([bq, bk, bv])[:, None, :]
    qkv = _linear(xs, ws, bs, 3)

    qkvh = (
        qkv.reshape(3, _B, _S, _H, _DH)
        .transpose(0, 1, 3, 2, 4)
        .reshape(3, _BH, _S, _DH)
    )
    cnt_T, _ = _consts()
    vcum = _sc_cumsum(qkvh)
    vc = _attn(qkvh, vcum.reshape(_BH, _S, _DH), jnp.asarray(cnt_T))

    vc2 = (
        vc.reshape(_B, _H, _S, _DH)
        .transpose(0, 2, 1, 3)
        .reshape(1, _B, _S, _D)
    )
    out = _linear(vc2, Wo.T[None], bo[None, None, :], 1)
    return out[0]


# speedup vs baseline: 1.0136x; 1.0007x over previous
import functools
import math

import jax
import jax.numpy as jnp
import numpy as np
from jax import lax
from jax.experimental import pallas as pl
from jax.experimental.pallas import tpu as pltpu
from jax.experimental.pallas import tpu_sc as plsc

_B, _S, _D, _H = 2, 2048, 768, 12
_DH = _D // _H
_BH = _B * _H
_RAND = 5 * int(np.ceil(np.log(_S)))
_TOP = 5 * int(np.log(_S))
_SCALE = 1.0 / math.sqrt(_DH)
_BLK = 256
_NBLK = _S // _BLK

def _threefry_pair(keypair, x0, x1):
    rot1 = (13, 15, 26, 6)
    rot2 = (17, 29, 16, 24)

    def rotl(x, r):
        return (x << np.uint32(r)) | (x >> np.uint32(32 - r))

    x0 = x0.astype(np.uint32).copy()
    x1 = x1.astype(np.uint32).copy()
    ks0, ks1 = np.uint32(keypair[0]), np.uint32(keypair[1])
    ks2 = ks0 ^ ks1 ^ np.uint32(0x1BD11BDA)
    sched = [(rot1, ks1, ks2), (rot2, ks2, ks0), (rot1, ks0, ks1),
             (rot2, ks1, ks2), (rot1, ks2, ks0)]
    with np.errstate(over="ignore"):
        x0 = x0 + ks0
        x1 = x1 + ks1
        for i, (rots, a0, a1) in enumerate(sched):
            for r in rots:
                x0 = x0 + x1
                x1 = rotl(x1, r) ^ x0
            x0 = x0 + a0
            x1 = x1 + a1 + np.uint32(i + 1)
    return x0, x1


def _rand_index():
    root = (np.uint32(0), np.uint32(42))
    z = np.zeros(2, np.uint32)
    b1, b2 = _threefry_pair(root, z, np.arange(2, dtype=np.uint32))
    child = (b1[1], b2[1])
    n = _S * _RAND
    o1, o2 = _threefry_pair(child, np.zeros(n, np.uint32),
                            np.arange(n, dtype=np.uint32))
    bits = o1 ^ o2
    return (bits % np.uint32(_S)).astype(np.int32).reshape(_S, _RAND)


_CONSTS: list = []


def _cnt_matrix():
    if not _CONSTS:
        ri = _rand_index()
        cnt_T = np.zeros((_S, _S), np.int8)
        np.add.at(cnt_T, (ri.ravel(), np.repeat(np.arange(_S), _RAND)), 1)
        _CONSTS.append(cnt_T)
    return _CONSTS[0]


def _linear_kern(x_ref, w_ref, b_ref, o_ref):
    o_ref[0, 0] = (
        jnp.dot(x_ref[0, 0], w_ref[0], preferred_element_type=jnp.float32)
        + b_ref[0]
    )


def _linear(xs, ws, bs, n_stack, sb=512):
    return pl.pallas_call(
        _linear_kern,
        grid=(n_stack, _B, _S // sb),
        in_specs=[
            pl.BlockSpec((1, 1, sb, _D), lambda i, b, s: (i, b, s, 0)),
            pl.BlockSpec((1, _D, _D), lambda i, b, s: (i, 0, 0)),
            pl.BlockSpec((1, 1, _D), lambda i, b, s: (i, 0, 0)),
        ],
        out_specs=pl.BlockSpec((1, 1, sb, _D), lambda i, b, s: (i, b, s, 0)),
        out_shape=jax.ShapeDtypeStruct((n_stack, _B, _S, _D), jnp.float32),
    )(xs, ws, bs)


_NT = (((1,), (1,)), ((), ()))


def _attn_kern(q_ref, k_ref, v_ref, vcum_ref, cnt_ref, o_ref, qi_ref, idx_ref):
    def blk_body(r, carry):
        smax, ssum = carry
        kb = k_ref[0, 0, pl.ds(r * _BLK, _BLK), :]
        at = jax.lax.dot_general(kb, q_ref[0, 0], _NT,
                                 preferred_element_type=jnp.float32)
        cf = cnt_ref[pl.ds(r * _BLK, _BLK), :].astype(jnp.float32)
        masked = jnp.where(cf > 0.0, at, -1e30)
        smax = jnp.maximum(smax, jnp.max(masked, axis=0, keepdims=True))
        ssum = ssum + jnp.sum(at * cf, axis=0, keepdims=True)
        return smax, ssum

    init = (jnp.full((1, _S), -1e30, jnp.float32), jnp.zeros((1, _S), jnp.float32))
    smax, ssum = jax.lax.fori_loop(0, _NBLK, blk_body, init)
    disc = smax - ssum / _S

    lane = jax.lax.broadcasted_iota(jnp.int32, (1, _S), 1)
    qi_ref[...] = jnp.zeros((_RAND, _DH), jnp.float32)

    def top_body(i, dcur):
        m = jnp.max(dcur)
        idx = jnp.min(jnp.where(dcur == m, lane, _S))
        idx_ref[i] = idx
        qi_ref[pl.ds(i, 1), :] = q_ref[0, 0, pl.ds(idx, 1), :]
        return jnp.where(lane == idx, -3e38, dcur)

    jax.lax.fori_loop(0, _TOP, top_body, disc)

    qi_ref[pl.ds(_TOP, _RAND - _TOP), :] = jnp.broadcast_to(
        qi_ref[pl.ds(0, 1), :], (_RAND - _TOP, _DH))

    qk = jax.lax.dot_general(qi_ref[...], k_ref[0, 0], _NT,
                             preferred_element_type=jnp.float32) * _SCALE
    m = jnp.max(qk, axis=1, keepdims=True)
    e = jnp.exp(qk - m)
    p = e / jnp.sum(e, axis=1, keepdims=True)
    upd = jnp.dot(p, v_ref[0, 0], preferred_element_type=jnp.float32)

    o_ref[0] = vcum_ref[0]
    qi_ref[...] = upd

    def scat_body(i, c):
        s = idx_ref[i]
        o_ref[0, pl.ds(s, 1), :] = qi_ref[pl.ds(i, 1), :]
        return c

    jax.lax.fori_loop(0, _TOP, scat_body, 0)


def _attn(qkvh, vcum, cntT):
    head_spec = lambda i: pl.BlockSpec(
        (1, 1, _S, _DH), lambda g, i=i: (i, g, 0, 0))
    return pl.pallas_call(
        _attn_kern,
        grid=(_BH,),
        in_specs=[
            head_spec(0),
            head_spec(1),
            head_spec(2),
            pl.BlockSpec((1, _S, _DH), lambda g: (g, 0, 0)),
            pl.BlockSpec((_S, _S), lambda g: (0, 0)),
        ],
        out_specs=pl.BlockSpec((1, _S, _DH), lambda g: (g, 0, 0)),
        out_shape=jax.ShapeDtypeStruct((_BH, _S, _DH), jnp.float32),
        scratch_shapes=[
            pltpu.VMEM((_RAND, _DH), jnp.float32),
            pltpu.SMEM((_RAND,), jnp.int32),
        ],
    )(qkvh, qkvh, qkvh, vcum, cntT)


_CHUNK = 512
_NCH = _S // _CHUNK


def _sc_cumsum(qkvh):
    mesh = plsc.VectorSubcoreMesh(core_axis_name="c", subcore_axis_name="s")

    @functools.partial(
        pl.kernel,
        mesh=mesh,
        out_type=jax.ShapeDtypeStruct((_BH * _S, _DH), jnp.float32),
        scratch_types=[
            pltpu.VMEM((_CHUNK, _DH), jnp.float32),
        ],
        compiler_params=pltpu.CompilerParams(use_tc_tiling_on_sc=False),
    )
    def cum_kernel(qkvh_hbm, out_hbm, chunk_v):
        wid = lax.axis_index("s") * 2 + lax.axis_index("c")

        @pl.when(wid < _BH)
        def _():
            bh = wid
            carry = tuple(jnp.zeros((16,), jnp.float32) for _ in range(4))
            for ch in range(_NCH):
                pltpu.sync_copy(
                    qkvh_hbm.at[2, bh, pl.ds(ch * _CHUNK, _CHUNK), :],
                    chunk_v)

                def body(i8, acc):
                    for r in range(8):
                        i = i8 * 8 + r
                        outs = []
                        for d in range(4):
                            a = acc[d] + chunk_v[i, pl.ds(d * 16, 16)]
                            chunk_v[i, pl.ds(d * 16, 16)] = a
                            outs.append(a)
                        acc = tuple(outs)
                    return acc

                carry = jax.lax.fori_loop(0, _CHUNK // 8, body, carry)
                pltpu.sync_copy(
                    chunk_v,
                    out_hbm.at[pl.ds((bh * _NCH + ch) * _CHUNK, _CHUNK), :])

    return cum_kernel(qkvh)


def kernel(queries, keys, values, Wq, bq, Wk, bk, Wv, bv, Wo, bo):
    xs = jnp.stack([queries, keys, values])
    ws = jnp.stack([Wq.T, Wk.T, Wv.T])
    bs = jnp.stack([bq, bk, bv])[:, None, :]
    qkv = _linear(xs, ws, bs, 3)

    qkvh = (
        qkv.reshape(3, _B, _S, _H, _DH)
        .transpose(0, 1, 3, 2, 4)
        .reshape(3, _BH, _S, _DH)
    )
    vcum = _sc_cumsum(qkvh)
    vc = _attn(qkvh, vcum.reshape(_BH, _S, _DH), jnp.asarray(_cnt_matrix()))

    vc2 = (
        vc.reshape(_B, _H, _S, _DH)
        .transpose(0, 2, 1, 3)
        .reshape(1, _B, _S, _D)
    )
    out = _linear(vc2, Wo.T[None], bo[None, None, :], 1)
    return out[0]


# speedup vs baseline: 1.0186x; 1.0050x over previous
import functools
import math

import jax
import jax.numpy as jnp
import numpy as np
from jax import lax
from jax.experimental import pallas as pl
from jax.experimental.pallas import tpu as pltpu
from jax.experimental.pallas import tpu_sc as plsc

_B, _S, _D, _H = 2, 2048, 768, 12
_DH = _D // _H
_BH = _B * _H
_RAND = 5 * int(np.ceil(np.log(_S)))
_TOP = 5 * int(np.log(_S))
_SCALE = 1.0 / math.sqrt(_DH)
_BLK = 256
_NBLK = _S // _BLK

def _threefry_pair(keypair, x0, x1):
    rot1 = (13, 15, 26, 6)
    rot2 = (17, 29, 16, 24)

    def rotl(x, r):
        return (x << np.uint32(r)) | (x >> np.uint32(32 - r))

    x0 = x0.astype(np.uint32).copy()
    x1 = x1.astype(np.uint32).copy()
    ks0, ks1 = np.uint32(keypair[0]), np.uint32(keypair[1])
    ks2 = ks0 ^ ks1 ^ np.uint32(0x1BD11BDA)
    sched = [(rot1, ks1, ks2), (rot2, ks2, ks0), (rot1, ks0, ks1),
             (rot2, ks1, ks2), (rot1, ks2, ks0)]
    with np.errstate(over="ignore"):
        x0 = x0 + ks0
        x1 = x1 + ks1
        for i, (rots, a0, a1) in enumerate(sched):
            for r in rots:
                x0 = x0 + x1
                x1 = rotl(x1, r) ^ x0
            x0 = x0 + a0
            x1 = x1 + a1 + np.uint32(i + 1)
    return x0, x1


def _rand_index():
    root = (np.uint32(0), np.uint32(42))
    z = np.zeros(2, np.uint32)
    b1, b2 = _threefry_pair(root, z, np.arange(2, dtype=np.uint32))
    child = (b1[1], b2[1])
    n = _S * _RAND
    o1, o2 = _threefry_pair(child, np.zeros(n, np.uint32),
                            np.arange(n, dtype=np.uint32))
    bits = o1 ^ o2
    return (bits % np.uint32(_S)).astype(np.int32).reshape(_S, _RAND)


_CONSTS: list = []


def _cnt_matrix():
    if not _CONSTS:
        ri = _rand_index()
        cnt_T = np.zeros((_S, _S), np.int8)
        np.add.at(cnt_T, (ri.ravel(), np.repeat(np.arange(_S), _RAND)), 1)
        _CONSTS.append(cnt_T)
    return _CONSTS[0]


def _linear_kern(x_ref, w_ref, b_ref, o_ref):
    o_ref[0, 0] = (
        jnp.dot(x_ref[0, 0], w_ref[0], preferred_element_type=jnp.float32)
        + b_ref[0]
    )


def _linear(xs, ws, bs, n_stack, sb=512):
    return pl.pallas_call(
        _linear_kern,
        grid=(n_stack, _B, _S // sb),
        in_specs=[
            pl.BlockSpec((1, 1, sb, _D), lambda i, b, s: (i, b, s, 0)),
            pl.BlockSpec((1, _D, _D), lambda i, b, s: (i, 0, 0)),
            pl.BlockSpec((1, 1, _D), lambda i, b, s: (i, 0, 0)),
        ],
        out_specs=pl.BlockSpec((1, 1, sb, _D), lambda i, b, s: (i, b, s, 0)),
        out_shape=jax.ShapeDtypeStruct((n_stack, _B, _S, _D), jnp.float32),
    )(xs, ws, bs)


_NT = (((1,), (1,)), ((), ()))


def _attn_kern(q_ref, k_ref, v_ref, vcum_ref, cnt_ref, o_ref, qi_ref, idx_ref):
    def blk_body(r, carry):
        smax, ssum = carry
        kb = k_ref[0, 0, pl.ds(r * _BLK, _BLK), :]
        at = jax.lax.dot_general(kb, q_ref[0, 0], _NT,
                                 preferred_element_type=jnp.float32)
        cf = cnt_ref[pl.ds(r * _BLK, _BLK), :].astype(jnp.float32)
        masked = jnp.where(cf > 0.0, at, -1e30)
        smax = jnp.maximum(smax, jnp.max(masked, axis=0, keepdims=True))
        ssum = ssum + jnp.sum(at * cf, axis=0, keepdims=True)
        return smax, ssum

    init = (jnp.full((1, _S), -1e30, jnp.float32), jnp.zeros((1, _S), jnp.float32))
    smax, ssum = jax.lax.fori_loop(0, _NBLK, blk_body, init)
    disc = smax - ssum / _S

    lane = jax.lax.broadcasted_iota(jnp.int32, (1, _S), 1)
    qi_ref[...] = jnp.zeros((_RAND, _DH), jnp.float32)

    def top_body(i, dcur):
        m = jnp.max(dcur)
        idx = jnp.min(jnp.where(dcur == m, lane, _S))
        idx_ref[i] = idx
        qi_ref[pl.ds(i, 1), :] = q_ref[0, 0, pl.ds(idx, 1), :]
        return jnp.where(lane == idx, -3e38, dcur)

    jax.lax.fori_loop(0, _TOP, top_body, disc)

    qi_ref[pl.ds(_TOP, _RAND - _TOP), :] = jnp.broadcast_to(
        qi_ref[pl.ds(0, 1), :], (_RAND - _TOP, _DH))

    qk = jax.lax.dot_general(qi_ref[...], k_ref[0, 0], _NT,
                             preferred_element_type=jnp.float32) * _SCALE
    m = jnp.max(qk, axis=1, keepdims=True)
    e = jnp.exp(qk - m)
    p = e / jnp.sum(e, axis=1, keepdims=True)
    upd = jnp.dot(p, v_ref[0, 0], preferred_element_type=jnp.float32)

    o_ref[0] = vcum_ref[0]
    qi_ref[...] = upd

    def scat_body(i, c):
        s = idx_ref[i]
        o_ref[0, pl.ds(s, 1), :] = qi_ref[pl.ds(i, 1), :]
        return c

    jax.lax.fori_loop(0, _TOP, scat_body, 0)


def _attn(qkvh, vcum, cntT):
    head_spec = lambda i: pl.BlockSpec(
        (1, 1, _S, _DH), lambda g, i=i: (i, g, 0, 0))
    return pl.pallas_call(
        _attn_kern,
        grid=(_BH,),
        in_specs=[
            head_spec(0),
            head_spec(1),
            head_spec(2),
            pl.BlockSpec((1, _S, _DH), lambda g: (g, 0, 0)),
            pl.BlockSpec((_S, _S), lambda g: (0, 0)),
        ],
        out_specs=pl.BlockSpec((1, _S, _DH), lambda g: (g, 0, 0)),
        out_shape=jax.ShapeDtypeStruct((_BH, _S, _DH), jnp.float32),
        scratch_shapes=[
            pltpu.VMEM((_RAND, _DH), jnp.float32),
            pltpu.SMEM((_RAND,), jnp.int32),
        ],
    )(qkvh, qkvh, qkvh, vcum, cntT)


_CHUNK = 512
_NCH = _S // _CHUNK


def _sc_cumsum(qkvh):
    mesh = plsc.VectorSubcoreMesh(core_axis_name="c", subcore_axis_name="s")

    @functools.partial(
        pl.kernel,
        mesh=mesh,
        out_type=jax.ShapeDtypeStruct((_BH * _S, _DH), jnp.float32),
        scratch_types=[
            pltpu.VMEM((_CHUNK, _DH), jnp.float32),
            pltpu.VMEM((_CHUNK, _DH), jnp.float32),
            pltpu.SemaphoreType.DMA,
            pltpu.SemaphoreType.DMA,
            pltpu.SemaphoreType.DMA,
            pltpu.SemaphoreType.DMA,
        ],
        compiler_params=pltpu.CompilerParams(use_tc_tiling_on_sc=False),
    )
    def cum_kernel(qkvh_hbm, out_hbm, c0, c1, li0, li1, lo0, lo1):
        wid = lax.axis_index("s") * 2 + lax.axis_index("c")
        bufs = (c0, c1)
        isems = (li0, li1)
        osems = (lo0, lo1)

        @pl.when(wid < _BH)
        def _():
            bh = wid
            carry = tuple(jnp.zeros((16,), jnp.float32) for _ in range(4))
            loads = [None] * _NCH
            stores = [None, None]
            loads[0] = pltpu.async_copy(
                qkvh_hbm.at[2, bh, pl.ds(0, _CHUNK), :], bufs[0], isems[0])
            for ch in range(_NCH):
                cur = bufs[ch % 2]
                if ch + 1 < _NCH:
                    nb = (ch + 1) % 2
                    if stores[nb] is not None:
                        stores[nb].wait()
                        stores[nb] = None
                    loads[ch + 1] = pltpu.async_copy(
                        qkvh_hbm.at[2, bh,
                                    pl.ds((ch + 1) * _CHUNK, _CHUNK), :],
                        bufs[nb], isems[nb])
                loads[ch].wait()

                def body(i8, acc, chunk_v=cur):
                    for r in range(8):
                        i = i8 * 8 + r
                        outs = []
                        for d in range(4):
                            a = acc[d] + chunk_v[i, pl.ds(d * 16, 16)]
                            chunk_v[i, pl.ds(d * 16, 16)] = a
                            outs.append(a)
                        acc = tuple(outs)
                    return acc

                carry = jax.lax.fori_loop(0, _CHUNK // 8, body, carry)
                stores[ch % 2] = pltpu.async_copy(
                    cur,
                    out_hbm.at[pl.ds((bh * _NCH + ch) * _CHUNK, _CHUNK), :],
                    osems[ch % 2])
            for st in stores:
                if st is not None:
                    st.wait()

    return cum_kernel(qkvh)


def kernel(queries, keys, values, Wq, bq, Wk, bk, Wv, bv, Wo, bo):
    xs = jnp.stack([queries, keys, values])
    ws = jnp.stack([Wq.T, Wk.T, Wv.T])
    bs = jnp.stack([bq, bk, bv])[:, None, :]
    qkv = _linear(xs, ws, bs, 3)

    qkvh = (
        qkv.reshape(3, _B, _S, _H, _DH)
        .transpose(0, 1, 3, 2, 4)
        .reshape(3, _BH, _S, _DH)
    )
    vcum = _sc_cumsum(qkvh)
    vc = _attn(qkvh, vcum.reshape(_BH, _S, _DH), jnp.asarray(_cnt_matrix()))

    vc2 = (
        vc.reshape(_B, _H, _S, _DH)
        .transpose(0, 2, 1, 3)
        .reshape(1, _B, _S, _D)
    )
    out = _linear(vc2, Wo.T[None], bo[None, None, :], 1)
    return out[0]
